# baseline (speedup 1.0000x reference)
import functools

import jax
import jax.numpy as jnp
from jax import lax
from jax.experimental import pallas as pl
from jax.experimental.pallas import tpu as pltpu
from jax.experimental.pallas import tpu_sc as plsc

_LANES = 16


def _sc_gather(user_ids, item_ids, utab2, itab2, D):
    B = user_ids.shape[0]
    P = 128 // D
    info = plsc.get_sparse_core_info()
    NC, NS = info.num_cores, info.num_subcores
    NW = NC * NS
    b_per_w = B // NW
    rows_per_w = b_per_w // P
    mesh = plsc.VectorSubcoreMesh(core_axis_name="c", subcore_axis_name="s")

    @functools.partial(
        pl.kernel,
        mesh=mesh,
        out_type=(
            jax.ShapeDtypeStruct((B // P, P * D), jnp.float32),
            jax.ShapeDtypeStruct((B // P, P * D), jnp.float32),
        ),
        scratch_types=[
            pltpu.VMEM((b_per_w,), jnp.int32),
            pltpu.VMEM((b_per_w,), jnp.int32),
            pltpu.VMEM((b_per_w // 2,), jnp.int32),
            pltpu.VMEM((b_per_w // 2,), jnp.int32),
            pltpu.VMEM((b_per_w // 2, 128), jnp.float32),
            pltpu.VMEM((b_per_w // 2, 128), jnp.float32),
            pltpu.VMEM((rows_per_w // 2, P * D), jnp.float32),
            pltpu.VMEM((rows_per_w // 2, P * D), jnp.float32),
            pltpu.SemaphoreType.DMA,
            pltpu.SemaphoreType.DMA,
        ],
    )
    def gk(uids_hbm, iids_hbm, utab_hbm, itab_hbm, u_out, i_out,
           uidx_v, iidx_v, udiv_v, idiv_v, urows_v, irows_v,
           upack_v, ipack_v, usem, isem):
        wid = lax.axis_index("s") * NC + lax.axis_index("c")
        base = wid * b_per_w
        half = b_per_w // 2
        hrows = rows_per_w // 2
        hgroups = half // _LANES
        pltpu.sync_copy(uids_hbm.at[pl.ds(base, b_per_w)], uidx_v)
        pltpu.sync_copy(iids_hbm.at[pl.ds(base, b_per_w)], iidx_v)

        for h in range(2):
            hoff = h * half

            def div_body(g, carry):
                goff = g * _LANES
                udiv_v[pl.ds(goff, _LANES)] = jnp.right_shift(
                    uidx_v[pl.ds(hoff + goff, _LANES)], 2)
                idiv_v[pl.ds(goff, _LANES)] = jnp.right_shift(
                    iidx_v[pl.ds(hoff + goff, _LANES)], 2)
                return carry

            lax.fori_loop(0, hgroups, div_body, 0)
            cu = pltpu.async_copy(utab_hbm.at[udiv_v], urows_v, usem)
            ci = pltpu.async_copy(itab_hbm.at[idiv_v], irows_v, isem)
            cu.wait()
            ci.wait()

            def pack_body(g, carry):
                goff = g * _LANES
                uvec = uidx_v[pl.ds(hoff + goff, _LANES)]
                ivec = iidx_v[pl.ds(hoff + goff, _LANES)]
                for l in range(_LANES):
                    j = goff + l
                    q = j // P
                    po = (j % P) * D
                    uo = (uvec[l] & (P - 1)) * D
                    io = (ivec[l] & (P - 1)) * D
                    for c in range(0, D, _LANES):
                        upack_v[q, pl.ds(po + c, _LANES)] = (
                            urows_v[j, pl.ds(uo + c, _LANES)])
                        ipack_v[q, pl.ds(po + c, _LANES)] = (
                            irows_v[j, pl.ds(io + c, _LANES)])
                return carry

            lax.fori_loop(0, hgroups, pack_body, 0)
            obase = wid * rows_per_w + h * hrows
            pltpu.sync_copy(upack_v, u_out.at[pl.ds(obase, hrows)])
            pltpu.sync_copy(ipack_v, i_out.at[pl.ds(obase, hrows)])

    return gk(user_ids, item_ids, utab2, itab2)


def _mlp_body(u_ref, i_ref, w1a_ref, w1b_ref, b1_ref, w2_ref, b2_ref,
              w3_ref, b3_ref, w4_ref, b4_ref, o_ref):
    h = jnp.dot(u_ref[...], w1a_ref[...], preferred_element_type=jnp.float32)
    h = h + jnp.dot(i_ref[...], w1b_ref[...], preferred_element_type=jnp.float32)
    h = jnp.maximum(h + b1_ref[...], 0.0)
    h = jnp.dot(h, w2_ref[...], preferred_element_type=jnp.float32) + b2_ref[...]
    h = jnp.maximum(h, 0.0)
    h = jnp.dot(h, w3_ref[...], preferred_element_type=jnp.float32) + b3_ref[...]
    h = jnp.maximum(h, 0.0)
    o_ref[...] = jnp.sum(h * w4_ref[...], axis=1) + b4_ref[0, 0]


def kernel(user_ids, item_ids, user_table, item_table,
           W1, b1, W2, b2, W3, b3, W4, b4):
    B = user_ids.shape[0]
    V, D = user_table.shape
    P = 128 // D
    utab2 = user_table.reshape(V // P, P * D)
    itab2 = item_table.reshape(V // P, P * D)
    u4, i4 = _sc_gather(user_ids, item_ids, utab2, itab2, D)
    u = u4.reshape(B, D)
    it = i4.reshape(B, D)
    out = pl.pallas_call(
        _mlp_body,
        out_shape=jax.ShapeDtypeStruct((B,), jnp.float32),
    )(u, it, W1[:D], W1[D:], b1.reshape(1, -1), W2, b2.reshape(1, -1),
      W3, b3.reshape(1, -1), W4.reshape(1, -1), b4.reshape(1, 1))
    return out


# speedup vs baseline: 1.4986x; 1.4986x over previous
import functools

import jax
import jax.numpy as jnp
from jax import lax
from jax.experimental import pallas as pl
from jax.experimental.pallas import tpu as pltpu
from jax.experimental.pallas import tpu_sc as plsc

_LANES = 16


def _sc_gather(user_ids, item_ids, user_table, item_table):
    B = user_ids.shape[0]
    D = user_table.shape[1]
    info = plsc.get_sparse_core_info()
    NC, NS = info.num_cores, info.num_subcores
    NW = NC * NS
    b_per_w = B // NW
    mesh = plsc.VectorSubcoreMesh(core_axis_name="c", subcore_axis_name="s")

    @functools.partial(
        pl.kernel,
        mesh=mesh,
        out_type=(
            jax.ShapeDtypeStruct((B, D), jnp.float32),
            jax.ShapeDtypeStruct((B, D), jnp.float32),
        ),
        scratch_types=[
            pltpu.VMEM((b_per_w,), jnp.int32),
            pltpu.VMEM((b_per_w,), jnp.int32),
            pltpu.VMEM((b_per_w // 2, D), jnp.float32),
            pltpu.VMEM((b_per_w // 2, D), jnp.float32),
            pltpu.SemaphoreType.DMA,
            pltpu.SemaphoreType.DMA,
            pltpu.SemaphoreType.DMA,
            pltpu.SemaphoreType.DMA,
            pltpu.SemaphoreType.DMA,
            pltpu.SemaphoreType.DMA,
            pltpu.SemaphoreType.DMA,
            pltpu.SemaphoreType.DMA,
        ],
    )
    def gk(uids_hbm, iids_hbm, utab_hbm, itab_hbm, u_out, i_out,
           uidx_v, iidx_v, urows_v, irows_v, usem, isem,
           usem1, isem1, usem2, isem2, usem3, isem3):
        usems = [usem, usem1, usem2, usem3]
        isems = [isem, isem1, isem2, isem3]
        wid = lax.axis_index("s") * NC + lax.axis_index("c")
        base = wid * b_per_w
        half = b_per_w // 2
        n_groups = half // _LANES
        pltpu.sync_copy(uids_hbm.at[pl.ds(base, b_per_w)], uidx_v)
        pltpu.sync_copy(iids_hbm.at[pl.ds(base, b_per_w)], iidx_v)
        lane_iota = lax.iota(jnp.int32, _LANES)

        for h in range(2):
            hoff = h * half

            def body(g, carry):
                goff = hoff + g * _LANES
                dbase = g * _LANES
                uvec = uidx_v[pl.ds(goff, _LANES)]
                ivec = iidx_v[pl.ds(goff, _LANES)]
                for l in range(_LANES):
                    ur = uvec[l]
                    ir = ivec[l]
                    pltpu.async_copy(utab_hbm.at[pl.ds(ur, 1), :],
                                     urows_v.at[pl.ds(dbase + l, 1), :],
                                     usems[l % 4])
                    pltpu.async_copy(itab_hbm.at[pl.ds(ir, 1), :],
                                     irows_v.at[pl.ds(dbase + l, 1), :],
                                     isems[l % 4])
                return carry

            lax.fori_loop(0, n_groups, body, 0)
            qrt = half // 4
            for k in range(4):
                pltpu.make_async_copy(utab_hbm.at[pl.ds(0, qrt), :],
                                      urows_v.at[pl.ds(0, qrt), :],
                                      usems[k]).wait()
                pltpu.make_async_copy(itab_hbm.at[pl.ds(0, qrt), :],
                                      irows_v.at[pl.ds(0, qrt), :],
                                      isems[k]).wait()
            pltpu.sync_copy(urows_v, u_out.at[pl.ds(base + hoff, half)])
            pltpu.sync_copy(irows_v, i_out.at[pl.ds(base + hoff, half)])

    return gk(user_ids, item_ids, user_table, item_table)


def _mlp_body(u_ref, i_ref, w1a_ref, w1b_ref, b1_ref, w2_ref, b2_ref,
              w3_ref, b3_ref, w4_ref, b4_ref, o_ref):
    h = jnp.dot(u_ref[...], w1a_ref[...], preferred_element_type=jnp.float32)
    h = h + jnp.dot(i_ref[...], w1b_ref[...], preferred_element_type=jnp.float32)
    h = jnp.maximum(h + b1_ref[...], 0.0)
    h = jnp.dot(h, w2_ref[...], preferred_element_type=jnp.float32) + b2_ref[...]
    h = jnp.maximum(h, 0.0)
    h = jnp.dot(h, w3_ref[...], preferred_element_type=jnp.float32) + b3_ref[...]
    h = jnp.maximum(h, 0.0)
    o_ref[...] = jnp.sum(h * w4_ref[...], axis=1) + b4_ref[0, 0]


def kernel(user_ids, item_ids, user_table, item_table,
           W1, b1, W2, b2, W3, b3, W4, b4):
    B = user_ids.shape[0]
    D = user_table.shape[1]
    u, it = _sc_gather(user_ids, item_ids, user_table, item_table)
    out = pl.pallas_call(
        _mlp_body,
        out_shape=jax.ShapeDtypeStruct((B,), jnp.float32),
    )(u, it, W1[:D], W1[D:], b1.reshape(1, -1), W2, b2.reshape(1, -1),
      W3, b3.reshape(1, -1), W4.reshape(1, -1), b4.reshape(1, 1))
    return out


# speedup vs baseline: 1.4998x; 1.0008x over previous
import functools

import jax
import jax.numpy as jnp
from jax import lax
from jax.experimental import pallas as pl
from jax.experimental.pallas import tpu as pltpu
from jax.experimental.pallas import tpu_sc as plsc

_LANES = 16


def _sc_gather(user_ids, item_ids, user_table, item_table):
    B = user_ids.shape[0]
    D = user_table.shape[1]
    info = plsc.get_sparse_core_info()
    NC, NS = info.num_cores, info.num_subcores
    NW = NC * NS
    b_per_w = B // NW
    mesh = plsc.VectorSubcoreMesh(core_axis_name="c", subcore_axis_name="s")

    @functools.partial(
        pl.kernel,
        mesh=mesh,
        out_type=(
            jax.ShapeDtypeStruct((B, D), jnp.float32),
            jax.ShapeDtypeStruct((B, D), jnp.float32),
        ),
        scratch_types=[
            pltpu.VMEM((b_per_w,), jnp.int32),
            pltpu.VMEM((b_per_w,), jnp.int32),
            pltpu.VMEM((b_per_w // 2, D), jnp.float32),
            pltpu.VMEM((b_per_w // 2, D), jnp.float32),
            pltpu.SemaphoreType.DMA,
            pltpu.SemaphoreType.DMA,
            pltpu.SemaphoreType.DMA,
            pltpu.SemaphoreType.DMA,
            pltpu.SemaphoreType.DMA,
            pltpu.SemaphoreType.DMA,
            pltpu.SemaphoreType.DMA,
            pltpu.SemaphoreType.DMA,
        ],
    )
    def gk(uids_hbm, iids_hbm, utab_hbm, itab_hbm, u_out, i_out,
           uidx_v, iidx_v, urows_v, irows_v, usem, isem,
           usem1, isem1, usem2, isem2, usem3, isem3):
        usems = [usem, usem1, usem2, usem3]
        isems = [isem, isem1, isem2, isem3]
        wid = lax.axis_index("s") * NC + lax.axis_index("c")
        base = wid * b_per_w
        half = b_per_w // 2
        n_groups = half // _LANES
        pltpu.sync_copy(uids_hbm.at[pl.ds(base, b_per_w)], uidx_v)
        pltpu.sync_copy(iids_hbm.at[pl.ds(base, b_per_w)], iidx_v)

        for h in range(2):
            hoff = h * half

            def body(g, carry):
                goff = hoff + g * _LANES
                dbase = g * _LANES
                uvec = uidx_v[pl.ds(goff, _LANES)]
                ivec = iidx_v[pl.ds(goff, _LANES)]
                for l in range(_LANES):
                    ur = uvec[l]
                    ir = ivec[l]
                    pltpu.async_copy(utab_hbm.at[pl.ds(ur, 1), :],
                                     urows_v.at[pl.ds(dbase + l, 1), :],
                                     usems[l % 4])
                    pltpu.async_copy(itab_hbm.at[pl.ds(ir, 1), :],
                                     irows_v.at[pl.ds(dbase + l, 1), :],
                                     isems[l % 4])
                return carry

            lax.fori_loop(0, n_groups, body, 0)
            qrt = half // 4
            for k in range(4):
                pltpu.make_async_copy(utab_hbm.at[pl.ds(0, qrt), :],
                                      urows_v.at[pl.ds(0, qrt), :],
                                      usems[k]).wait()
                pltpu.make_async_copy(itab_hbm.at[pl.ds(0, qrt), :],
                                      irows_v.at[pl.ds(0, qrt), :],
                                      isems[k]).wait()
            pltpu.sync_copy(urows_v, u_out.at[pl.ds(base + hoff, half)])
            pltpu.sync_copy(irows_v, i_out.at[pl.ds(base + hoff, half)])

    return gk(user_ids, item_ids, user_table, item_table)


def _mlp_body(u_ref, i_ref, w1a_ref, w1b_ref, b1_ref, w2_ref, b2_ref,
              w3_ref, b3_ref, w4_ref, b4_ref, o_ref):
    h = jnp.dot(u_ref[...], w1a_ref[...], preferred_element_type=jnp.float32)
    h = h + jnp.dot(i_ref[...], w1b_ref[...], preferred_element_type=jnp.float32)
    h = jnp.maximum(h + b1_ref[...], 0.0)
    h = jnp.dot(h, w2_ref[...], preferred_element_type=jnp.float32) + b2_ref[...]
    h = jnp.maximum(h, 0.0)
    h = jnp.dot(h, w3_ref[...], preferred_element_type=jnp.float32) + b3_ref[...]
    h = jnp.maximum(h, 0.0)
    o_ref[...] = jnp.sum(h * w4_ref[...], axis=1) + b4_ref[0, 0]


def kernel(user_ids, item_ids, user_table, item_table,
           W1, b1, W2, b2, W3, b3, W4, b4):
    B = user_ids.shape[0]
    D = user_table.shape[1]
    u, it = _sc_gather(user_ids, item_ids, user_table, item_table)
    out = pl.pallas_call(
        _mlp_body,
        out_shape=jax.ShapeDtypeStruct((B,), jnp.float32),
    )(u, it, W1[:D], W1[D:], b1.reshape(1, -1), W2, b2.reshape(1, -1),
      W3, b3.reshape(1, -1), W4.reshape(1, -1), b4.reshape(1, 1))
    return out
